# baseline (speedup 1.0000x reference)
import functools

import jax
import jax.numpy as jnp
from jax import lax
from jax.experimental import pallas as pl
from jax.experimental.pallas import tpu as pltpu
from jax.experimental.pallas import tpu_sc as plsc

NC = 2
NS = 16
L = 16
TEXW = 1024
CH = 16
B = 128
GROUPS = B // L
SLABS_PER_TILE = TEXW // NS


def _tex_pool_sc(n):
    pts_per_tile = n // NS
    iters = pts_per_tile // B
    mesh = plsc.VectorSubcoreMesh(
        core_axis_name="c", subcore_axis_name="s",
        num_cores=NC, num_subcores=NS)

    @functools.partial(
        pl.kernel,
        out_type=jax.ShapeDtypeStruct((n, NC * CH), jnp.float32),
        mesh=mesh,
        compiler_params=pltpu.CompilerParams(use_tc_tiling_on_sc=False,
                                             needs_layout_passes=False),
        scratch_types=[
            pltpu.HBM((NC, TEXW * TEXW, CH), jnp.float32),
            pltpu.VMEM((CH * TEXW,), jnp.float32),
            pltpu.VMEM((TEXW, CH), jnp.float32),
            pltpu.VMEM((B,), jnp.float32),
            pltpu.VMEM((B,), jnp.float32),
            pltpu.VMEM((4, B), jnp.int32),
            pltpu.VMEM((4, B), jnp.float32),
            pltpu.VMEM((4, B, CH), jnp.float32),
            pltpu.VMEM((B, CH), jnp.float32),
            pltpu.SemaphoreType.DMA,
        ],
    )
    def k(u_hbm, v_hbm, t0, t1, out_hbm, tab, slab_v, tr_v, u_v, v_v,
          idx_v, w_v, g_v, o_v, sem):
        c = lax.axis_index("c")
        s = lax.axis_index("s")
        lanes_x = lax.iota(jnp.int32, L) * TEXW

        def relayout(t_hbm):
            mytab = tab.at[c]

            @pl.loop(0, SLABS_PER_TILE)
            def _slab(i):
                y = s * SLABS_PER_TILE + i
                cps = [
                    pltpu.async_copy(
                        t_hbm.at[y, ch],
                        slab_v.at[pl.ds(ch * TEXW, TEXW)], sem)
                    for ch in range(CH)
                ]
                for cp in cps:
                    cp.wait()

                @pl.loop(0, TEXW, unroll=8)
                def _tx(x):
                    tr_v[x, :] = plsc.load_gather(slab_v, [lanes_x + x])

                pltpu.sync_copy(tr_v, mytab.at[pl.ds(y * TEXW, TEXW)])

        @pl.when(c == 0)
        def _():
            relayout(t0)

        @pl.when(c == 1)
        def _():
            relayout(t1)

        plsc.subcore_barrier()

        mytab = tab.at[c]
        base_t = s * pts_per_tile

        @pl.loop(0, iters)
        def _iter(it):
            base = base_t + it * B
            pltpu.sync_copy(u_hbm.at[pl.ds(base, B)], u_v)
            pltpu.sync_copy(v_hbm.at[pl.ds(base, B)], v_v)
            for g in range(GROUPS):
                sl = pl.ds(g * L, L)
                u = u_v[sl] * float(TEXW - 1)
                v = v_v[sl] * float(TEXW - 1)
                x0 = u.astype(jnp.int32)
                y0 = v.astype(jnp.int32)
                x0 = jnp.minimum(jnp.maximum(x0, 0), TEXW - 1)
                y0 = jnp.minimum(jnp.maximum(y0, 0), TEXW - 1)
                x1 = jnp.minimum(x0 + 1, TEXW - 1)
                y1 = jnp.minimum(y0 + 1, TEXW - 1)
                wx = u - x0.astype(jnp.float32)
                wy = v - y0.astype(jnp.float32)
                r0 = y0 << 10
                r1 = y1 << 10
                idx_v[0, sl] = r0 + x0
                idx_v[1, sl] = r0 + x1
                idx_v[2, sl] = r1 + x0
                idx_v[3, sl] = r1 + x1
                cx = 1.0 - wx
                cy = 1.0 - wy
                w_v[0, sl] = cx * cy
                w_v[1, sl] = wx * cy
                w_v[2, sl] = cx * wy
                w_v[3, sl] = wx * wy
            cps = [
                pltpu.async_copy(mytab.at[idx_v.at[c4]], g_v.at[c4], sem)
                for c4 in range(4)
            ]
            for cp in cps:
                cp.wait()
            for g in range(GROUPS):
                sl = pl.ds(g * L, L)
                w00 = w_v[0, sl]
                w01 = w_v[1, sl]
                w10 = w_v[2, sl]
                w11 = w_v[3, sl]
                for j in range(L):
                    p = g * L + j
                    o_v[p, :] = (g_v[0, p, :] * w00[j] + g_v[1, p, :] * w01[j]
                                 + g_v[2, p, :] * w10[j]
                                 + g_v[3, p, :] * w11[j])
            pltpu.sync_copy(
                o_v, out_hbm.at[pl.ds(base, B), pl.ds(c * CH, CH)])

    return k


def kernel(uv, tex0, tex1):
    n = uv.shape[0]
    u = uv[:, 0]
    v = uv[:, 1]
    t0 = tex0.transpose(0, 2, 1)
    t1 = tex1.transpose(0, 2, 1)
    return _tex_pool_sc(n)(u, v, t0, t1)


# speedup vs baseline: 1.4062x; 1.4062x over previous
import functools

import jax
import jax.numpy as jnp
from jax import lax
from jax.experimental import pallas as pl
from jax.experimental.pallas import tpu as pltpu
from jax.experimental.pallas import tpu_sc as plsc

NC = 2
NS = 16
L = 16
TEXW = 1024
CH = 16
B = 128
GROUPS = B // L
SLABS_PER_TILE = TEXW // NS


def _tex_pool_sc(n):
    pts_per_tile = n // NS
    iters = pts_per_tile // B
    mesh = plsc.VectorSubcoreMesh(
        core_axis_name="c", subcore_axis_name="s",
        num_cores=NC, num_subcores=NS)

    @functools.partial(
        pl.kernel,
        out_type=jax.ShapeDtypeStruct((n, NC * CH), jnp.float32),
        mesh=mesh,
        compiler_params=pltpu.CompilerParams(use_tc_tiling_on_sc=False,
                                             needs_layout_passes=False),
        scratch_types=[
            pltpu.HBM((NC, TEXW * TEXW, CH), jnp.float32),
            pltpu.VMEM((CH * (TEXW + 8),), jnp.float32),
            pltpu.VMEM((TEXW, CH), jnp.float32),
            pltpu.VMEM((B,), jnp.float32),
            pltpu.VMEM((B,), jnp.float32),
            pltpu.VMEM((4, B), jnp.int32),
            pltpu.VMEM((4, B), jnp.float32),
            pltpu.VMEM((4, B, CH), jnp.float32),
            pltpu.VMEM((B, CH), jnp.float32),
            pltpu.SemaphoreType.DMA,
        ],
    )
    def k(u_hbm, v_hbm, t0, t1, out_hbm, tab, slab_v, tr_v, u_v, v_v,
          idx_v, w_v, g_v, o_v, sem):
        c = lax.axis_index("c")
        s = lax.axis_index("s")
        lanes_x = lax.iota(jnp.int32, L) * (TEXW + 8)

        def relayout(t_hbm):
            mytab = tab.at[c]

            @pl.loop(0, SLABS_PER_TILE)
            def _slab(i):
                y = s * SLABS_PER_TILE + i
                cps = [
                    pltpu.async_copy(
                        t_hbm.at[y, ch],
                        slab_v.at[pl.ds(ch * (TEXW + 8), TEXW)], sem)
                    for ch in range(CH)
                ]
                for cp in cps:
                    cp.wait()

                @pl.loop(0, TEXW, unroll=8)
                def _tx(x):
                    tr_v[x, :] = plsc.load_gather(slab_v, [lanes_x + x])

                pltpu.sync_copy(tr_v, mytab.at[pl.ds(y * TEXW, TEXW)])

        @pl.when(c == 0)
        def _():
            relayout(t0)

        @pl.when(c == 1)
        def _():
            relayout(t1)

        plsc.subcore_barrier()

        mytab = tab.at[c]
        base_t = s * pts_per_tile

        @pl.loop(0, iters)
        def _iter(it):
            base = base_t + it * B
            pltpu.sync_copy(u_hbm.at[pl.ds(base, B)], u_v)
            pltpu.sync_copy(v_hbm.at[pl.ds(base, B)], v_v)
            for g in range(GROUPS):
                sl = pl.ds(g * L, L)
                u = u_v[sl] * float(TEXW - 1)
                v = v_v[sl] * float(TEXW - 1)
                x0 = u.astype(jnp.int32)
                y0 = v.astype(jnp.int32)
                x0 = jnp.minimum(jnp.maximum(x0, 0), TEXW - 1)
                y0 = jnp.minimum(jnp.maximum(y0, 0), TEXW - 1)
                x1 = jnp.minimum(x0 + 1, TEXW - 1)
                y1 = jnp.minimum(y0 + 1, TEXW - 1)
                wx = u - x0.astype(jnp.float32)
                wy = v - y0.astype(jnp.float32)
                r0 = y0 << 10
                r1 = y1 << 10
                idx_v[0, sl] = r0 + x0
                idx_v[1, sl] = r0 + x1
                idx_v[2, sl] = r1 + x0
                idx_v[3, sl] = r1 + x1
                cx = 1.0 - wx
                cy = 1.0 - wy
                w_v[0, sl] = cx * cy
                w_v[1, sl] = wx * cy
                w_v[2, sl] = cx * wy
                w_v[3, sl] = wx * wy
            cps = [
                pltpu.async_copy(mytab.at[idx_v.at[c4]], g_v.at[c4], sem)
                for c4 in range(4)
            ]
            for cp in cps:
                cp.wait()
            for g in range(GROUPS):
                sl = pl.ds(g * L, L)
                w00 = w_v[0, sl]
                w01 = w_v[1, sl]
                w10 = w_v[2, sl]
                w11 = w_v[3, sl]
                for j in range(L):
                    p = g * L + j
                    o_v[p, :] = (g_v[0, p, :] * w00[j] + g_v[1, p, :] * w01[j]
                                 + g_v[2, p, :] * w10[j]
                                 + g_v[3, p, :] * w11[j])
            pltpu.sync_copy(
                o_v, out_hbm.at[pl.ds(base, B), pl.ds(c * CH, CH)])

    return k


def kernel(uv, tex0, tex1):
    n = uv.shape[0]
    u = uv[:, 0]
    v = uv[:, 1]
    t0 = tex0.transpose(0, 2, 1)
    t1 = tex1.transpose(0, 2, 1)
    return _tex_pool_sc(n)(u, v, t0, t1)


# speedup vs baseline: 1.7629x; 1.2537x over previous
import functools

import jax
import jax.numpy as jnp
from jax import lax
from jax.experimental import pallas as pl
from jax.experimental.pallas import tpu as pltpu
from jax.experimental.pallas import tpu_sc as plsc

NC = 2
NS = 16
L = 16
TEXW = 1024
SSTR = TEXW + 8
CH = 16
B = 128
GROUPS = B // L
SLABS_PER_TILE = TEXW // NS


def _tex_pool_sc(n):
    pts_per_tile = n // NS
    iters = pts_per_tile // B
    mesh = plsc.VectorSubcoreMesh(
        core_axis_name="c", subcore_axis_name="s",
        num_cores=NC, num_subcores=NS)

    @functools.partial(
        pl.kernel,
        out_type=jax.ShapeDtypeStruct((n, NC * CH), jnp.float32),
        mesh=mesh,
        compiler_params=pltpu.CompilerParams(use_tc_tiling_on_sc=False,
                                             needs_layout_passes=False),
        scratch_types=[
            pltpu.HBM((NC, TEXW * TEXW, CH), jnp.float32),
            pltpu.VMEM((2, CH * SSTR), jnp.float32),
            pltpu.VMEM((2, TEXW, CH), jnp.float32),
            pltpu.VMEM((2, B), jnp.float32),
            pltpu.VMEM((2, B), jnp.float32),
            pltpu.VMEM((2, 4, B), jnp.int32),
            pltpu.VMEM((2, 4, B), jnp.float32),
            pltpu.VMEM((2, 4, B, CH), jnp.float32),
            pltpu.VMEM((2, B, CH), jnp.float32),
            pltpu.SemaphoreType.DMA,
            pltpu.SemaphoreType.DMA,
            pltpu.SemaphoreType.DMA,
        ],
    )
    def k(u_hbm, v_hbm, t0, t1, out_hbm, tab, slab_v, tr_v, u_v, v_v,
          idx_v, w_v, g_v, o_v, sem_in, sem_g, sem_out):
        c = lax.axis_index("c")
        s = lax.axis_index("s")
        lanes_x = lax.iota(jnp.int32, L) * SSTR

        def relayout(t_hbm):
            mytab = tab.at[c]
            y0 = s * SLABS_PER_TILE

            def fire_in(i, buf):
                for ch in range(CH):
                    pltpu.async_copy(
                        t_hbm.at[y0 + i, ch],
                        slab_v.at[buf, pl.ds(ch * SSTR, TEXW)], sem_in)

            def drain_in(buf):
                for ch in range(CH):
                    pltpu.make_async_copy(
                        t_hbm.at[y0, ch],
                        slab_v.at[buf, pl.ds(ch * SSTR, TEXW)],
                        sem_in).wait()

            def drain_out():
                pltpu.make_async_copy(
                    tr_v.at[0], mytab.at[pl.ds(0, TEXW)], sem_out).wait()

            fire_in(0, 0)

            @pl.loop(0, SLABS_PER_TILE)
            def _slab(i):
                buf = lax.rem(i, 2)

                @pl.when(i + 1 < SLABS_PER_TILE)
                def _():
                    fire_in(i + 1, 1 - buf)

                drain_in(buf)

                @pl.when(i >= 2)
                def _():
                    drain_out()

                @pl.loop(0, TEXW, unroll=8)
                def _tx(x):
                    tr_v[buf, x, :] = plsc.load_gather(
                        slab_v.at[buf], [lanes_x + x])

                pltpu.async_copy(
                    tr_v.at[buf], mytab.at[pl.ds((y0 + i) * TEXW, TEXW)],
                    sem_out)

            drain_out()
            drain_out()

        @pl.when(c == 0)
        def _():
            relayout(t0)

        @pl.when(c == 1)
        def _():
            relayout(t1)

        plsc.subcore_barrier()

        mytab = tab.at[c]
        base_t = s * pts_per_tile

        def fire_uv(i, buf):
            base = base_t + i * B
            pltpu.async_copy(u_hbm.at[pl.ds(base, B)], u_v.at[buf], sem_in)
            pltpu.async_copy(v_hbm.at[pl.ds(base, B)], v_v.at[buf], sem_in)

        def drain_uv(buf):
            pltpu.make_async_copy(u_hbm.at[pl.ds(0, B)], u_v.at[buf],
                                  sem_in).wait()
            pltpu.make_async_copy(v_hbm.at[pl.ds(0, B)], v_v.at[buf],
                                  sem_in).wait()

        def drain_gathers(buf):
            for c4 in range(4):
                pltpu.make_async_copy(mytab.at[idx_v.at[buf, c4]],
                                      g_v.at[buf, c4], sem_g).wait()

        def drain_out2(buf):
            pltpu.make_async_copy(
                o_v.at[buf],
                out_hbm.at[pl.ds(0, B), pl.ds(c * CH, CH)], sem_out).wait()

        fire_uv(0, 0)

        @pl.loop(0, iters + 1)
        def _iter(i):
            buf = lax.rem(i, 2)

            @pl.when(i < iters)
            def _prep():
                drain_uv(buf)
                for g in range(GROUPS):
                    sl = pl.ds(g * L, L)
                    u = u_v[buf, sl] * float(TEXW - 1)
                    v = v_v[buf, sl] * float(TEXW - 1)
                    x0 = u.astype(jnp.int32)
                    y0 = v.astype(jnp.int32)
                    x0 = jnp.minimum(jnp.maximum(x0, 0), TEXW - 1)
                    y0 = jnp.minimum(jnp.maximum(y0, 0), TEXW - 1)
                    x1 = jnp.minimum(x0 + 1, TEXW - 1)
                    y1 = jnp.minimum(y0 + 1, TEXW - 1)
                    wx = u - x0.astype(jnp.float32)
                    wy = v - y0.astype(jnp.float32)
                    r0 = y0 << 10
                    r1 = y1 << 10
                    idx_v[buf, 0, sl] = r0 + x0
                    idx_v[buf, 1, sl] = r0 + x1
                    idx_v[buf, 2, sl] = r1 + x0
                    idx_v[buf, 3, sl] = r1 + x1
                    cx = 1.0 - wx
                    cy = 1.0 - wy
                    w_v[buf, 0, sl] = cx * cy
                    w_v[buf, 1, sl] = wx * cy
                    w_v[buf, 2, sl] = cx * wy
                    w_v[buf, 3, sl] = wx * wy
                for c4 in range(4):
                    pltpu.async_copy(mytab.at[idx_v.at[buf, c4]],
                                     g_v.at[buf, c4], sem_g)

                @pl.when(i + 1 < iters)
                def _():
                    fire_uv(i + 1, 1 - buf)

            @pl.when(i > 0)
            def _blend():
                pbuf = lax.rem(i + 1, 2)

                @pl.when(i >= 3)
                def _():
                    drain_out2(pbuf)

                drain_gathers(pbuf)
                for g in range(GROUPS):
                    sl = pl.ds(g * L, L)
                    w00 = w_v[pbuf, 0, sl]
                    w01 = w_v[pbuf, 1, sl]
                    w10 = w_v[pbuf, 2, sl]
                    w11 = w_v[pbuf, 3, sl]
                    for j in range(L):
                        p = g * L + j
                        o_v[pbuf, p, :] = (g_v[pbuf, 0, p, :] * w00[j]
                                           + g_v[pbuf, 1, p, :] * w01[j]
                                           + g_v[pbuf, 2, p, :] * w10[j]
                                           + g_v[pbuf, 3, p, :] * w11[j])
                base = base_t + (i - 1) * B
                pltpu.async_copy(
                    o_v.at[pbuf],
                    out_hbm.at[pl.ds(base, B), pl.ds(c * CH, CH)], sem_out)

        drain_out2(0)
        drain_out2(1)

    return k


def kernel(uv, tex0, tex1):
    n = uv.shape[0]
    u = uv[:, 0]
    v = uv[:, 1]
    t0 = tex0.transpose(0, 2, 1)
    t1 = tex1.transpose(0, 2, 1)
    return _tex_pool_sc(n)(u, v, t0, t1)


# speedup vs baseline: 1.9212x; 1.0898x over previous
import functools

import jax
import jax.numpy as jnp
from jax import lax
from jax.experimental import pallas as pl
from jax.experimental.pallas import tpu as pltpu
from jax.experimental.pallas import tpu_sc as plsc

NC = 2
NS = 16
L = 16
TEXW = 1024
SSTR = TEXW + 8
CH = 16
B = 128
GROUPS = B // L
SLABS_PER_TILE = TEXW // NS


def _tex_pool_sc(n):
    pts_per_tile = n // NS
    iters = pts_per_tile // B
    mesh = plsc.VectorSubcoreMesh(
        core_axis_name="c", subcore_axis_name="s",
        num_cores=NC, num_subcores=NS)

    @functools.partial(
        pl.kernel,
        out_type=jax.ShapeDtypeStruct((n, NC * CH), jnp.float32),
        mesh=mesh,
        compiler_params=pltpu.CompilerParams(use_tc_tiling_on_sc=False,
                                             needs_layout_passes=False),
        scratch_types=[
            pltpu.HBM((NC, TEXW * TEXW, CH), jnp.float32),
            pltpu.VMEM((2, CH, SSTR), jnp.float32),
            pltpu.VMEM((2, TEXW, CH), jnp.float32),
            pltpu.VMEM((2, B), jnp.float32),
            pltpu.VMEM((2, B), jnp.float32),
            pltpu.VMEM((2, 4, B), jnp.int32),
            pltpu.VMEM((2, 4, B), jnp.float32),
            pltpu.VMEM((2, 4, B, CH), jnp.float32),
            pltpu.VMEM((2, B, CH), jnp.float32),
            pltpu.SemaphoreType.DMA,
            pltpu.SemaphoreType.DMA,
            pltpu.SemaphoreType.DMA,
        ],
    )
    def k(u_hbm, v_hbm, t0, t1, out_hbm, tab, slab_v, tr_v, u_v, v_v,
          idx_v, w_v, g_v, o_v, sem_in, sem_g, sem_out):
        c = lax.axis_index("c")
        s = lax.axis_index("s")
        lanes_ch = lax.iota(jnp.int32, L)

        def relayout(t_hbm):
            mytab = tab.at[c]
            y0 = s * SLABS_PER_TILE

            def fire_in(i, buf):
                for chh in range(2):
                    for xh in range(8):
                        pltpu.async_copy(
                            t_hbm.at[2 * (y0 + i) + chh, xh],
                            slab_v.at[buf, pl.ds(chh * 8, 8),
                                      pl.ds(xh * 128, 128)], sem_in)

            def drain_in(buf):
                for chh in range(2):
                    for xh in range(8):
                        pltpu.make_async_copy(
                            t_hbm.at[chh, xh],
                            slab_v.at[buf, pl.ds(chh * 8, 8),
                                      pl.ds(xh * 128, 128)],
                            sem_in).wait()

            def drain_out():
                pltpu.make_async_copy(
                    tr_v.at[0], mytab.at[pl.ds(0, TEXW)], sem_out).wait()

            fire_in(0, 0)

            @pl.loop(0, SLABS_PER_TILE)
            def _slab(i):
                buf = lax.rem(i, 2)

                @pl.when(i + 1 < SLABS_PER_TILE)
                def _():
                    fire_in(i + 1, 1 - buf)

                drain_in(buf)

                @pl.when(i >= 2)
                def _():
                    drain_out()

                @pl.loop(0, TEXW, unroll=8)
                def _tx(x):
                    xs = jnp.zeros((L,), jnp.int32) + x
                    tr_v[buf, x, :] = plsc.load_gather(
                        slab_v.at[buf], [lanes_ch, xs])

                pltpu.async_copy(
                    tr_v.at[buf], mytab.at[pl.ds((y0 + i) * TEXW, TEXW)],
                    sem_out)

            drain_out()
            drain_out()

        @pl.when(c == 0)
        def _():
            relayout(t0)

        @pl.when(c == 1)
        def _():
            relayout(t1)

        plsc.subcore_barrier()

        mytab = tab.at[c]
        base_t = s * pts_per_tile

        def fire_uv(i, buf):
            base = base_t + i * B
            pltpu.async_copy(u_hbm.at[pl.ds(base, B)], u_v.at[buf], sem_in)
            pltpu.async_copy(v_hbm.at[pl.ds(base, B)], v_v.at[buf], sem_in)

        def drain_uv(buf):
            pltpu.make_async_copy(u_hbm.at[pl.ds(0, B)], u_v.at[buf],
                                  sem_in).wait()
            pltpu.make_async_copy(v_hbm.at[pl.ds(0, B)], v_v.at[buf],
                                  sem_in).wait()

        def drain_gathers(buf):
            for c4 in range(4):
                pltpu.make_async_copy(mytab.at[idx_v.at[buf, c4]],
                                      g_v.at[buf, c4], sem_g).wait()

        def drain_out2(buf):
            pltpu.make_async_copy(
                o_v.at[buf],
                out_hbm.at[pl.ds(0, B), pl.ds(c * CH, CH)], sem_out).wait()

        fire_uv(0, 0)

        @pl.loop(0, iters + 1)
        def _iter(i):
            buf = lax.rem(i, 2)

            @pl.when(i < iters)
            def _prep():
                drain_uv(buf)
                for g in range(GROUPS):
                    sl = pl.ds(g * L, L)
                    u = u_v[buf, sl] * float(TEXW - 1)
                    v = v_v[buf, sl] * float(TEXW - 1)
                    x0 = u.astype(jnp.int32)
                    y0 = v.astype(jnp.int32)
                    x0 = jnp.minimum(jnp.maximum(x0, 0), TEXW - 1)
                    y0 = jnp.minimum(jnp.maximum(y0, 0), TEXW - 1)
                    x1 = jnp.minimum(x0 + 1, TEXW - 1)
                    y1 = jnp.minimum(y0 + 1, TEXW - 1)
                    wx = u - x0.astype(jnp.float32)
                    wy = v - y0.astype(jnp.float32)
                    r0 = y0 << 10
                    r1 = y1 << 10
                    idx_v[buf, 0, sl] = r0 + x0
                    idx_v[buf, 1, sl] = r0 + x1
                    idx_v[buf, 2, sl] = r1 + x0
                    idx_v[buf, 3, sl] = r1 + x1
                    cx = 1.0 - wx
                    cy = 1.0 - wy
                    w_v[buf, 0, sl] = cx * cy
                    w_v[buf, 1, sl] = wx * cy
                    w_v[buf, 2, sl] = cx * wy
                    w_v[buf, 3, sl] = wx * wy
                for c4 in range(4):
                    pltpu.async_copy(mytab.at[idx_v.at[buf, c4]],
                                     g_v.at[buf, c4], sem_g)

                @pl.when(i + 1 < iters)
                def _():
                    fire_uv(i + 1, 1 - buf)

            @pl.when(i > 0)
            def _blend():
                pbuf = lax.rem(i + 1, 2)

                @pl.when(i >= 3)
                def _():
                    drain_out2(pbuf)

                drain_gathers(pbuf)
                for g in range(GROUPS):
                    sl = pl.ds(g * L, L)
                    w00 = w_v[pbuf, 0, sl]
                    w01 = w_v[pbuf, 1, sl]
                    w10 = w_v[pbuf, 2, sl]
                    w11 = w_v[pbuf, 3, sl]
                    for j in range(L):
                        p = g * L + j
                        o_v[pbuf, p, :] = (g_v[pbuf, 0, p, :] * w00[j]
                                           + g_v[pbuf, 1, p, :] * w01[j]
                                           + g_v[pbuf, 2, p, :] * w10[j]
                                           + g_v[pbuf, 3, p, :] * w11[j])
                base = base_t + (i - 1) * B
                pltpu.async_copy(
                    o_v.at[pbuf],
                    out_hbm.at[pl.ds(base, B), pl.ds(c * CH, CH)], sem_out)

        drain_out2(0)
        drain_out2(1)

    return k


def _native_view(tex):
    return (tex.reshape(TEXW, 8, 128, 2, 8)
            .transpose(0, 3, 1, 4, 2)
            .reshape(2 * TEXW, 8, 8, 128))


def kernel(uv, tex0, tex1):
    n = uv.shape[0]
    u = uv[:, 0]
    v = uv[:, 1]
    return _tex_pool_sc(n)(u, v, _native_view(tex0), _native_view(tex1))
